# baseline (speedup 1.0000x reference)
import functools

import jax
import jax.numpy as jnp
from jax import lax
from jax.experimental import pallas as pl
from jax.experimental.pallas import tpu as pltpu
from jax.experimental.pallas import tpu_sc as plsc

_B = 4
_T = 4096
_D = 1024
_NW = 32
_ROWS_PER_W = (_B * _T) // _NW
_C = 32
_NCHUNK = _ROWS_PER_W // _C
_LANES = 16
_SL = _D // _LANES


def _emb_body(ids_hbm, tok_hbm, pos_hbm, out_hbm, idx_v, tokbuf, posbuf, sem):
    wid = lax.axis_index("s") * 2 + lax.axis_index("c")
    base = wid * _ROWS_PER_W
    pos_base = lax.rem(base, _T)

    def chunk(ci, _):
        off = ci * _C
        pltpu.sync_copy(ids_hbm.at[pl.ds(base + off, _C)], idx_v)
        cp = pltpu.async_copy(tok_hbm.at[idx_v], tokbuf, sem)
        pltpu.sync_copy(pos_hbm.at[pl.ds(pos_base + off, _C)], posbuf)
        cp.wait()

        def add_row(r, _):
            for j in range(_SL):
                sl = pl.ds(j * _LANES, _LANES)
                posbuf[r, sl] = posbuf[r, sl] + tokbuf[r, sl]
            return 0

        lax.fori_loop(0, _C, add_row, 0)
        pltpu.sync_copy(posbuf, out_hbm.at[pl.ds(base + off, _C)])
        return 0

    lax.fori_loop(0, _NCHUNK, chunk, 0)


@jax.jit
def _emb(ids_flat, token_emb, pos_emb):
    mesh = plsc.VectorSubcoreMesh(core_axis_name="c", subcore_axis_name="s")
    call = functools.partial(
        pl.kernel,
        mesh=mesh,
        out_type=jax.ShapeDtypeStruct((_B * _T, _D), jnp.float32),
        scratch_types=[
            pltpu.VMEM((_C,), jnp.int32),
            pltpu.VMEM((_C, _D), jnp.float32),
            pltpu.VMEM((_C, _D), jnp.float32),
            pltpu.SemaphoreType.DMA,
        ],
    )(_emb_body)
    return call(ids_flat, token_emb, pos_emb)


def kernel(input_ids, token_emb, pos_emb):
    ids_flat = input_ids.astype(jnp.int32).reshape(-1)
    out = _emb(ids_flat, token_emb, pos_emb)
    return out.reshape(_B, _T, _D)


# speedup vs baseline: 1.4337x; 1.4337x over previous
import functools

import jax
import jax.numpy as jnp
from jax import lax
from jax.experimental import pallas as pl
from jax.experimental.pallas import tpu as pltpu
from jax.experimental.pallas import tpu_sc as plsc

_B = 4
_T = 4096
_D = 1024
_NW = 32
_TW = _T // _NW
_C = 32
_NTC = _TW // _C
_NJOB = _B * _NTC
_LANES = 16
_SL = _D // _LANES


def _emb_body(ids_hbm, tok_hbm, pos_hbm, out_hbm,
              idx_v, posbuf, tok0, tok1, sg0, sg1, ss0, ss1):
    wid = lax.axis_index("s") * 2 + lax.axis_index("c")
    t0 = wid * _TW

    for b in range(_B):
        pltpu.sync_copy(ids_hbm.at[pl.ds(b * _T + t0, _TW)],
                        idx_v.at[pl.ds(b * _TW, _TW)])
    pltpu.sync_copy(pos_hbm.at[pl.ds(t0, _C)], posbuf)

    toks = (tok0, tok1)
    sgs = (sg0, sg1)
    sss = (ss0, ss1)

    pltpu.async_copy(tok_hbm.at[idx_v.at[pl.ds(0, _C)]], tok0, sg0)

    def two_jobs(iv, _):
        for u in range(2):
            i = iv * 2 + u
            tok, sg, ss = toks[u], sgs[u], sss[u]
            otok, osg, oss = toks[1 - u], sgs[1 - u], sss[1 - u]

            @pl.when(i >= 1)
            def _():
                pltpu.make_async_copy(otok, out_hbm.at[pl.ds(0, _C)], oss).wait()

            @pl.when(i <= _NJOB - 2)
            def _():
                ni = i + 1
                ioff = lax.rem(ni, _B) * _TW + lax.div(ni, _B) * _C
                pltpu.async_copy(tok_hbm.at[idx_v.at[pl.ds(ioff, _C)]],
                                 otok, osg)

            pltpu.make_async_copy(tok_hbm.at[idx_v.at[pl.ds(0, _C)]],
                                  tok, sg).wait()

            def add_row(r, _):
                for j in range(_SL):
                    sl = pl.ds(j * _LANES, _LANES)
                    tok[r, sl] = tok[r, sl] + posbuf[r, sl]
                return 0

            lax.fori_loop(0, _C, add_row, 0)

            @pl.when(jnp.logical_and(lax.rem(i, _B) == _B - 1,
                                     i <= _NJOB - 2))
            def _():
                pltpu.sync_copy(
                    pos_hbm.at[pl.ds(t0 + (lax.div(i, _B) + 1) * _C, _C)],
                    posbuf)

            row = lax.rem(i, _B) * _T + t0 + lax.div(i, _B) * _C
            pltpu.async_copy(tok, out_hbm.at[pl.ds(row, _C)], ss)
        return 0

    lax.fori_loop(0, _NJOB // 2, two_jobs, 0)
    pltpu.make_async_copy(tok1, out_hbm.at[pl.ds(0, _C)], ss1).wait()


@jax.jit
def _emb(ids_flat, token_emb, pos_emb):
    mesh = plsc.VectorSubcoreMesh(core_axis_name="c", subcore_axis_name="s")
    call = functools.partial(
        pl.kernel,
        mesh=mesh,
        out_type=jax.ShapeDtypeStruct((_B * _T, _D), jnp.float32),
        scratch_types=[
            pltpu.VMEM((_B * _TW,), jnp.int32),
            pltpu.VMEM((_C, _D), jnp.float32),
            pltpu.VMEM((_C, _D), jnp.float32),
            pltpu.VMEM((_C, _D), jnp.float32),
            pltpu.SemaphoreType.DMA,
            pltpu.SemaphoreType.DMA,
            pltpu.SemaphoreType.DMA,
            pltpu.SemaphoreType.DMA,
        ],
    )(_emb_body)
    return call(ids_flat, token_emb, pos_emb)


def kernel(input_ids, token_emb, pos_emb):
    ids_flat = input_ids.astype(jnp.int32).reshape(-1)
    out = _emb(ids_flat, token_emb, pos_emb)
    return out.reshape(_B, _T, _D)


# speedup vs baseline: 1.5558x; 1.0852x over previous
import functools

import jax
import jax.numpy as jnp
from jax import lax
from jax.experimental import pallas as pl
from jax.experimental.pallas import tpu as pltpu
from jax.experimental.pallas import tpu_sc as plsc

_B = 4
_T = 4096
_D = 1024
_NW = 32
_TW = _T // _NW
_C = 32
_NTC = _TW // _C
_NJOB = _B * _NTC
_LANES = 16
_SL = _D // _LANES


def _emb_body(ids_hbm, tok_hbm, pos_hbm, out_hbm,
              idx_v, posbuf, tok0, tok1, sg0, sg1, ss0, ss1):
    wid = lax.axis_index("s") * 2 + lax.axis_index("c")
    t0 = wid * _TW

    for b in range(_B):
        pltpu.sync_copy(ids_hbm.at[pl.ds(b * _T + t0, _TW)],
                        idx_v.at[pl.ds(b * _TW, _TW)])
    pltpu.sync_copy(pos_hbm.at[pl.ds(t0, _C)], posbuf)

    toks = (tok0, tok1)
    sgs = (sg0, sg1)
    sss = (ss0, ss1)

    pltpu.async_copy(tok_hbm.at[idx_v.at[pl.ds(0, _C)]], tok0, sg0)

    def two_jobs(iv, _):
        for u in range(2):
            i = iv * 2 + u
            tok, sg, ss = toks[u], sgs[u], sss[u]
            otok, osg, oss = toks[1 - u], sgs[1 - u], sss[1 - u]

            @pl.when(i >= 1)
            def _():
                pltpu.make_async_copy(otok, out_hbm.at[pl.ds(0, _C)], oss).wait()

            @pl.when(i <= _NJOB - 2)
            def _():
                ni = i + 1
                ioff = lax.rem(ni, _B) * _TW + lax.div(ni, _B) * _C
                pltpu.async_copy(tok_hbm.at[idx_v.at[pl.ds(ioff, _C)]],
                                 otok, osg)

            pltpu.make_async_copy(tok_hbm.at[idx_v.at[pl.ds(0, _C)]],
                                  tok, sg).wait()

            def add_row(r, _):
                for j in range(_SL):
                    sl = pl.ds(j * _LANES, _LANES)
                    plsc.addupdate(tok.at[r, sl], posbuf[r, sl])
                return 0

            lax.fori_loop(0, _C, add_row, 0)

            @pl.when(jnp.logical_and(lax.rem(i, _B) == _B - 1,
                                     i <= _NJOB - 2))
            def _():
                pltpu.sync_copy(
                    pos_hbm.at[pl.ds(t0 + (lax.div(i, _B) + 1) * _C, _C)],
                    posbuf)

            row = lax.rem(i, _B) * _T + t0 + lax.div(i, _B) * _C
            pltpu.async_copy(tok, out_hbm.at[pl.ds(row, _C)], ss)
        return 0

    lax.fori_loop(0, _NJOB // 2, two_jobs, 0)
    pltpu.make_async_copy(tok1, out_hbm.at[pl.ds(0, _C)], ss1).wait()


@jax.jit
def _emb(ids_flat, token_emb, pos_emb):
    mesh = plsc.VectorSubcoreMesh(core_axis_name="c", subcore_axis_name="s")
    call = functools.partial(
        pl.kernel,
        mesh=mesh,
        out_type=jax.ShapeDtypeStruct((_B * _T, _D), jnp.float32),
        scratch_types=[
            pltpu.VMEM((_B * _TW,), jnp.int32),
            pltpu.VMEM((_C, _D), jnp.float32),
            pltpu.VMEM((_C, _D), jnp.float32),
            pltpu.VMEM((_C, _D), jnp.float32),
            pltpu.SemaphoreType.DMA,
            pltpu.SemaphoreType.DMA,
            pltpu.SemaphoreType.DMA,
            pltpu.SemaphoreType.DMA,
        ],
    )(_emb_body)
    return call(ids_flat, token_emb, pos_emb)


def kernel(input_ids, token_emb, pos_emb):
    ids_flat = input_ids.astype(jnp.int32).reshape(-1)
    out = _emb(ids_flat, token_emb, pos_emb)
    return out.reshape(_B, _T, _D)


# speedup vs baseline: 1.9985x; 1.2845x over previous
import functools

import jax
import jax.numpy as jnp
from jax import lax
from jax.experimental import pallas as pl
from jax.experimental.pallas import tpu as pltpu
from jax.experimental.pallas import tpu_sc as plsc

_B = 4
_T = 4096
_D = 1024
_NW = 32
_TW = _T // _NW
_C = 16
_NTC = _TW // _C
_NJOB = _B * _NTC
_LANES = 16
_SL = _D // _LANES


def _job_coords(i):
    b = lax.rem(i, _B)
    tc = lax.div(i, _B)
    return b, tc


def _emb_body(ids_hbm, tok_hbm, pos_hbm, out_hbm,
              idx_v, pos0, pos1, tk0, tk1, tk2, tk3,
              sp0, sp1, sg0, sg1, sg2, sg3, ss0, ss1, ss2, ss3):
    wid = lax.axis_index("s") * 2 + lax.axis_index("c")
    t0 = wid * _TW

    toks = (tk0, tk1, tk2, tk3)
    sgs = (sg0, sg1, sg2, sg3)
    sss = (ss0, ss1, ss2, ss3)
    poss = (pos0, pos1)
    sps = (sp0, sp1)

    for b in range(_B):
        pltpu.sync_copy(ids_hbm.at[pl.ds(b * _T + t0, _TW)],
                        idx_v.at[pl.ds(b * _TW, _TW)])

    def fire_gather(i, buf, sem):
        b, tc = _job_coords(i)
        ioff = b * _TW + tc * _C
        pltpu.async_copy(tok_hbm.at[idx_v.at[pl.ds(ioff, _C)]], buf, sem)

    def fire_pos(p, buf, sem):
        pltpu.async_copy(pos_hbm.at[pl.ds(t0 + p * _C, _C)], buf, sem)

    fire_pos(0, pos0, sp0)
    fire_gather(0, tk0, sg0)
    fire_gather(1, tk1, sg1)

    def eight_jobs(iv, _):
        for u in range(8):
            i = iv * 8 + u
            k = u % 4
            pb = u // 4
            tok, sg, ss = toks[k], sgs[k], sss[k]

            @pl.when(i >= 2)
            def _():
                pltpu.make_async_copy(toks[(k + 2) % 4],
                                      out_hbm.at[pl.ds(0, _C)],
                                      sss[(k + 2) % 4]).wait()

            @pl.when(i <= _NJOB - 3)
            def _():
                fire_gather(i + 2, toks[(k + 2) % 4], sgs[(k + 2) % 4])

            if u == 0:
                pltpu.make_async_copy(pos_hbm.at[pl.ds(0, _C)],
                                      poss[0], sps[0]).wait()
                fire_pos(2 * iv + 1, poss[1], sps[1])
            if u == 4:
                pltpu.make_async_copy(pos_hbm.at[pl.ds(0, _C)],
                                      poss[1], sps[1]).wait()

                @pl.when(2 * iv + 2 <= _NTC - 1)
                def _():
                    fire_pos(2 * iv + 2, poss[0], sps[0])

            pltpu.make_async_copy(tok_hbm.at[idx_v.at[pl.ds(0, _C)]],
                                  tok, sg).wait()
            posb = poss[pb]

            def add_row(r, _):
                for j in range(_SL):
                    sl = pl.ds(j * _LANES, _LANES)
                    plsc.addupdate(tok.at[r, sl], posb[r, sl])
                return 0

            lax.fori_loop(0, _C, add_row, 0)

            b, tc = _job_coords(i)
            row = b * _T + t0 + tc * _C
            pltpu.async_copy(tok, out_hbm.at[pl.ds(row, _C)], ss)
        return 0

    lax.fori_loop(0, _NJOB // 8, eight_jobs, 0)
    pltpu.make_async_copy(tk2, out_hbm.at[pl.ds(0, _C)], ss2).wait()
    pltpu.make_async_copy(tk3, out_hbm.at[pl.ds(0, _C)], ss3).wait()


@jax.jit
def _emb(ids_flat, token_emb, pos_emb):
    mesh = plsc.VectorSubcoreMesh(core_axis_name="c", subcore_axis_name="s")
    call = functools.partial(
        pl.kernel,
        mesh=mesh,
        out_type=jax.ShapeDtypeStruct((_B * _T, _D), jnp.float32),
        scratch_types=[
            pltpu.VMEM((_B * _TW,), jnp.int32),
            pltpu.VMEM((_C, _D), jnp.float32),
            pltpu.VMEM((_C, _D), jnp.float32),
            pltpu.VMEM((_C, _D), jnp.float32),
            pltpu.VMEM((_C, _D), jnp.float32),
            pltpu.VMEM((_C, _D), jnp.float32),
            pltpu.VMEM((_C, _D), jnp.float32),
            pltpu.SemaphoreType.DMA,
            pltpu.SemaphoreType.DMA,
            pltpu.SemaphoreType.DMA,
            pltpu.SemaphoreType.DMA,
            pltpu.SemaphoreType.DMA,
            pltpu.SemaphoreType.DMA,
            pltpu.SemaphoreType.DMA,
            pltpu.SemaphoreType.DMA,
            pltpu.SemaphoreType.DMA,
            pltpu.SemaphoreType.DMA,
        ],
    )(_emb_body)
    return call(ids_flat, token_emb, pos_emb)


def kernel(input_ids, token_emb, pos_emb):
    ids_flat = input_ids.astype(jnp.int32).reshape(-1)
    out = _emb(ids_flat, token_emb, pos_emb)
    return out.reshape(_B, _T, _D)
